# Initial kernel scaffold; baseline (speedup 1.0000x reference)
import jax
import jax.numpy as jnp
from jax.experimental import pallas as pl


def kernel(x, table):
    raise NotImplementedError("write your pallas kernel here")


# speedup vs baseline: 9.2186x; 9.2186x over previous
import functools

import jax
import jax.numpy as jnp
from jax import lax
from jax.experimental import pallas as pl
from jax.experimental.pallas import tpu as pltpu
from jax.experimental.pallas import tpu_sc as plsc

NC = 2
NS = 16
NW = NC * NS
CHUNK = 128


@functools.partial(jax.jit, static_argnums=(2, 3))
def _gather_flat(idx, table, n, d):
    per_w = n // NW
    n_chunks = per_w // CHUNK
    idx3 = idx.reshape(NW, n_chunks, CHUNK)

    mesh = plsc.VectorSubcoreMesh(
        core_axis_name="c", subcore_axis_name="s",
        num_cores=NC, num_subcores=NS)

    @functools.partial(
        pl.kernel,
        out_type=jax.ShapeDtypeStruct((n, d), jnp.float32),
        mesh=mesh,
        scratch_types=[
            pltpu.VMEM((n_chunks, CHUNK), jnp.int32),
            pltpu.VMEM((2, CHUNK, d), jnp.float32),
            pltpu.SemaphoreType.DMA,
            pltpu.SemaphoreType.DMA,
        ],
    )
    def emb(idx_hbm, table_hbm, out_hbm, idx_v, rows_v, sem0, sem1):
        wid = lax.axis_index("s") * NC + lax.axis_index("c")
        base = wid * per_w
        pltpu.sync_copy(idx_hbm.at[wid], idx_v)
        sems = (sem0, sem1)
        for b in range(2):
            pltpu.async_copy(table_hbm.at[idx_v.at[b]], rows_v.at[b], sems[b])

        def step(g):
            for b in range(2):
                j = g * 2 + b
                pltpu.make_async_copy(
                    table_hbm.at[idx_v.at[j]], rows_v.at[b], sems[b]).wait()
                pltpu.sync_copy(
                    rows_v.at[b], out_hbm.at[pl.ds(base + j * CHUNK, CHUNK)])

                @pl.when(j + 2 < n_chunks)
                def _():
                    pltpu.async_copy(
                        table_hbm.at[idx_v.at[j + 2]], rows_v.at[b], sems[b])

        pl.loop(0, n_chunks // 2)(step)

    return emb(idx3, table)


def kernel(x, table):
    b, s = x.shape
    v, d = table.shape
    n = b * s
    flat = _gather_flat(x.reshape(n), table, n, d)
    return flat.reshape(b, s, d)


# speedup vs baseline: 9.2534x; 1.0038x over previous
import functools

import jax
import jax.numpy as jnp
from jax import lax
from jax.experimental import pallas as pl
from jax.experimental.pallas import tpu as pltpu
from jax.experimental.pallas import tpu_sc as plsc

NC = 2
NS = 16
NW = NC * NS
CHUNK = 128


@functools.partial(jax.jit, static_argnums=(2, 3))
def _gather_flat(idx, table, n, d):
    per_w = n // NW
    n_chunks = per_w // CHUNK
    idx3 = idx.reshape(NW, n_chunks, CHUNK)

    mesh = plsc.VectorSubcoreMesh(
        core_axis_name="c", subcore_axis_name="s",
        num_cores=NC, num_subcores=NS)

    @functools.partial(
        pl.kernel,
        out_type=jax.ShapeDtypeStruct((n, d), jnp.float32),
        mesh=mesh,
        scratch_types=[
            pltpu.VMEM((n_chunks, CHUNK), jnp.int32),
            pltpu.VMEM((4, CHUNK, d), jnp.float32),
            [pltpu.SemaphoreType.DMA] * 4,
            [pltpu.SemaphoreType.DMA] * 4,
        ],
    )
    def emb(idx_hbm, table_hbm, out_hbm, idx_v, rows_v, gsems, wsems):
        wid = lax.axis_index("s") * NC + lax.axis_index("c")
        base = wid * per_w
        pltpu.sync_copy(idx_hbm.at[wid], idx_v)
        for j in range(2):
            pltpu.async_copy(table_hbm.at[idx_v.at[j]], rows_v.at[j], gsems[j])

        def step(g):
            for b in range(4):
                j = g * 4 + b
                nb = (b + 2) % 4

                @pl.when(j >= 2)
                def _():
                    pltpu.make_async_copy(
                        rows_v.at[nb],
                        out_hbm.at[pl.ds(base + (j - 2) * CHUNK, CHUNK)],
                        wsems[nb]).wait()

                @pl.when(j + 2 < n_chunks)
                def _():
                    pltpu.async_copy(
                        table_hbm.at[idx_v.at[j + 2]], rows_v.at[nb],
                        gsems[nb])

                pltpu.make_async_copy(
                    table_hbm.at[idx_v.at[j]], rows_v.at[b], gsems[b]).wait()
                pltpu.async_copy(
                    rows_v.at[b], out_hbm.at[pl.ds(base + j * CHUNK, CHUNK)],
                    wsems[b])

        pl.loop(0, n_chunks // 4)(step)
        for j in (n_chunks - 2, n_chunks - 1):
            b = j % 4
            pltpu.make_async_copy(
                rows_v.at[b], out_hbm.at[pl.ds(base + j * CHUNK, CHUNK)],
                wsems[b]).wait()

    return emb(idx3, table)


def kernel(x, table):
    b, s = x.shape
    v, d = table.shape
    n = b * s
    flat = _gather_flat(x.reshape(n), table, n, d)
    return flat.reshape(b, s, d)


# speedup vs baseline: 9.2852x; 1.0034x over previous
import functools

import jax
import jax.numpy as jnp
from jax import lax
from jax.experimental import pallas as pl
from jax.experimental.pallas import tpu as pltpu
from jax.experimental.pallas import tpu_sc as plsc

NC = 2
NS = 16
NW = NC * NS
CHUNK = 128
NSLOT = 6
LOOK = 3


@functools.partial(jax.jit, static_argnums=(2, 3))
def _gather_flat(idx, table, n, d):
    per_w = n // NW
    n_chunks = per_w // CHUNK
    n_main = (n_chunks // NSLOT) * NSLOT
    idx3 = idx.reshape(NW, n_chunks, CHUNK)

    mesh = plsc.VectorSubcoreMesh(
        core_axis_name="c", subcore_axis_name="s",
        num_cores=NC, num_subcores=NS)

    @functools.partial(
        pl.kernel,
        out_type=jax.ShapeDtypeStruct((n, d), jnp.float32),
        mesh=mesh,
        scratch_types=[
            pltpu.VMEM((n_chunks, CHUNK), jnp.int32),
            pltpu.VMEM((NSLOT, CHUNK, d), jnp.float32),
            [pltpu.SemaphoreType.DMA] * NSLOT,
            [pltpu.SemaphoreType.DMA] * NSLOT,
        ],
    )
    def emb(idx_hbm, table_hbm, out_hbm, idx_v, rows_v, gsems, wsems):
        wid = lax.axis_index("s") * NC + lax.axis_index("c")
        base = wid * per_w

        def gather(j, slot):
            pltpu.async_copy(
                table_hbm.at[idx_v.at[j]], rows_v.at[slot], gsems[slot])

        def gather_wait(j, slot):
            pltpu.make_async_copy(
                table_hbm.at[idx_v.at[j]], rows_v.at[slot],
                gsems[slot]).wait()

        def write(j, slot):
            pltpu.async_copy(
                rows_v.at[slot], out_hbm.at[pl.ds(base + j * CHUNK, CHUNK)],
                wsems[slot])

        def write_wait(j, slot):
            pltpu.make_async_copy(
                rows_v.at[slot], out_hbm.at[pl.ds(base + j * CHUNK, CHUNK)],
                wsems[slot]).wait()

        pltpu.sync_copy(idx_hbm.at[wid], idx_v)
        for j in range(LOOK):
            gather(j, j)

        def step(g):
            for b in range(NSLOT):
                j = g * NSLOT + b
                nb = (b + LOOK) % NSLOT

                @pl.when(j >= LOOK)
                def _():
                    write_wait(j - LOOK, nb)

                @pl.when(j + LOOK < n_chunks)
                def _():
                    gather(j + LOOK, nb)

                gather_wait(j, b)
                write(j, b)

        pl.loop(0, n_main // NSLOT)(step)
        for j in range(n_main, n_chunks):
            gather_wait(j, j % NSLOT)
            write(j, j % NSLOT)
        for j in range(n_chunks - LOOK - (n_chunks - n_main), n_chunks):
            write_wait(j, j % NSLOT)

    return emb(idx3, table)


def kernel(x, table):
    b, s = x.shape
    v, d = table.shape
    n = b * s
    flat = _gather_flat(x.reshape(n), table, n, d)
    return flat.reshape(b, s, d)


# speedup vs baseline: 9.3399x; 1.0059x over previous
import functools

import jax
import jax.numpy as jnp
from jax import lax
from jax.experimental import pallas as pl
from jax.experimental.pallas import tpu as pltpu
from jax.experimental.pallas import tpu_sc as plsc

NC = 2
NS = 16
NW = NC * NS
CHUNK = 80
NSLOT = 8
LOOK = 5
WDRAIN = 3


@functools.partial(jax.jit, static_argnums=(2, 3))
def _gather_flat(idx, table, n, d):
    per_w = n // NW
    n_chunks = per_w // CHUNK
    n_main = (n_chunks // NSLOT) * NSLOT
    idx3 = idx.reshape(NW, n_chunks, CHUNK)

    mesh = plsc.VectorSubcoreMesh(
        core_axis_name="c", subcore_axis_name="s",
        num_cores=NC, num_subcores=NS)

    @functools.partial(
        pl.kernel,
        out_type=jax.ShapeDtypeStruct((n, d), jnp.float32),
        mesh=mesh,
        scratch_types=[
            pltpu.VMEM((n_chunks, CHUNK), jnp.int32),
            pltpu.VMEM((NSLOT, CHUNK, d), jnp.float32),
            [pltpu.SemaphoreType.DMA] * NSLOT,
            [pltpu.SemaphoreType.DMA] * NSLOT,
        ],
    )
    def emb(idx_hbm, table_hbm, out_hbm, idx_v, rows_v, gsems, wsems):
        wid = lax.axis_index("s") * NC + lax.axis_index("c")
        base = wid * per_w

        def gather(j, slot):
            pltpu.async_copy(
                table_hbm.at[idx_v.at[j]], rows_v.at[slot], gsems[slot])

        def gather_wait(j, slot):
            pltpu.make_async_copy(
                table_hbm.at[idx_v.at[j]], rows_v.at[slot],
                gsems[slot]).wait()

        def write(j, slot):
            pltpu.async_copy(
                rows_v.at[slot], out_hbm.at[pl.ds(base + j * CHUNK, CHUNK)],
                wsems[slot])

        def write_wait(j, slot):
            pltpu.make_async_copy(
                rows_v.at[slot], out_hbm.at[pl.ds(base + j * CHUNK, CHUNK)],
                wsems[slot]).wait()

        pltpu.sync_copy(idx_hbm.at[wid], idx_v)
        for j in range(LOOK):
            gather(j, j)

        def step(g):
            for b in range(NSLOT):
                j = g * NSLOT + b
                nb = (b + LOOK) % NSLOT

                @pl.when(j >= WDRAIN)
                def _():
                    write_wait(j - WDRAIN, (b + NSLOT - WDRAIN) % NSLOT)

                @pl.when(j + LOOK < n_chunks)
                def _():
                    gather(j + LOOK, nb)

                gather_wait(j, b)
                write(j, b)

        pl.loop(0, n_main // NSLOT)(step)
        for j in range(n_main, n_chunks):
            gather_wait(j, j % NSLOT)
            write(j, j % NSLOT)
        for j in range(n_chunks - WDRAIN, n_chunks):
            write_wait(j, j % NSLOT)

    return emb(idx3, table)


def kernel(x, table):
    b, s = x.shape
    v, d = table.shape
    n = b * s
    flat = _gather_flat(x.reshape(n), table, n, d)
    return flat.reshape(b, s, d)
